# baseline (speedup 1.0000x reference)
import functools

import jax
import jax.numpy as jnp
from jax import lax
from jax.experimental import pallas as pl
from jax.experimental.pallas import tpu as pltpu
from jax.experimental.pallas import tpu_sc as plsc

N = 2_000_000


def _build(NB=4, GS=512, NW=32, PTS_W=65_536, WB=2048, WC=1024, CHUNK=128,
           interpret=False):
    N_P = NW * PTS_W
    CPB = (NB * GS * GS) // NW
    SH = CPB.bit_length() - 1
    SHA = SH - (GS.bit_length() - 1)
    N_ALLOC = N_P + 8 * NW + WC
    NLANE = 16
    NBIN = NW + 1

    mesh = plsc.VectorSubcoreMesh(core_axis_name="c", subcore_axis_name="s",
                                  num_cores=2, num_subcores=16)
    params = pltpu.CompilerParams(needs_layout_passes=False,
                                  use_tc_tiling_on_sc=False)

    def _wid():
        return lax.axis_index("s") * 2 + lax.axis_index("c")

    iota = lambda: lax.iota(jnp.int32, 16)

    @functools.partial(
        pl.kernel,
        out_type=jax.ShapeDtypeStruct((NW, NLANE * NBIN), jnp.int32),
        mesh=mesh,
        compiler_params=params,
        interpret=interpret,
        scratch_types=[
            pltpu.VMEM((NLANE * NBIN,), jnp.int32),
            pltpu.VMEM((WB,), jnp.int32),
            pltpu.VMEM((WB,), jnp.int32),
        ],
    )
    def phase_a(b_hbm, r_hbm, counts_hbm, cnt_v, bwin, rwin):
        wid = _wid()
        io = iota()
        zero16 = jnp.zeros((16,), jnp.int32)

        def z(k, carry):
            cnt_v[pl.ds(k * 16, 16)] = zero16
            return carry

        lax.fori_loop(0, NBIN, z, 0)

        def win(g, carry):
            base = wid * PTS_W + g * WB
            pltpu.sync_copy(b_hbm.at[pl.ds(base, WB)], bwin)
            pltpu.sync_copy(r_hbm.at[pl.ds(base, WB)], rwin)

            def vr(k, c2):
                bv = bwin[pl.ds(k * 16, 16)]
                rv = rwin[pl.ds(k * 16, 16)]
                binv = (bv * GS + rv) >> SHA
                pidx = io * NBIN + binv
                cur = plsc.load_gather(cnt_v, [pidx])
                plsc.store_scatter(cnt_v, [pidx], cur + 1)
                return c2

            return lax.fori_loop(0, WB // 16, vr, carry)

        lax.fori_loop(0, PTS_W // WB, win, 0)
        pltpu.sync_copy(cnt_v, counts_hbm.at[wid])

    _o = jax.ShapeDtypeStruct((N_ALLOC,), jnp.int32)

    @functools.partial(
        pl.kernel,
        out_type=(_o, _o, _o, _o),
        mesh=mesh,
        compiler_params=params,
        interpret=interpret,
        scratch_types=[
            pltpu.VMEM((NLANE * NBIN,), jnp.int32),
            pltpu.VMEM((WB,), jnp.int32),
            pltpu.VMEM((WB,), jnp.int32),
            pltpu.VMEM((WB,), jnp.int32),
            pltpu.VMEM((2 * WB,), jnp.float32),
            pltpu.VMEM((WB,), jnp.int32),
            pltpu.VMEM((WB,), jnp.int32),
            pltpu.VMEM((WB,), jnp.int32),
            pltpu.VMEM((WB,), jnp.int32),
            pltpu.VMEM((WB,), jnp.int32),
            pltpu.SemaphoreType.DMA,
        ],
    )
    def phase_b(b_hbm, r_hbm, c_hbm, reg_hbm, bases_hbm,
                cell_hbm, i_hbm, r0_hbm, r1_hbm,
                ptr_v, bwin, rwin, cwin, regwin,
                cellbuf, ibuf, r0buf, r1buf, destbuf, sem):
        wid = _wid()
        io = iota()

        pltpu.sync_copy(bases_hbm.at[wid], ptr_v)

        def win(g, carry):
            base = wid * PTS_W + g * WB
            pltpu.sync_copy(b_hbm.at[pl.ds(base, WB)], bwin)
            pltpu.sync_copy(r_hbm.at[pl.ds(base, WB)], rwin)
            pltpu.sync_copy(c_hbm.at[pl.ds(base, WB)], cwin)
            pltpu.sync_copy(reg_hbm.at[pl.ds(2 * base, 2 * WB)], regwin)

            def vr(k, c2):
                loc = k * 16
                bv = bwin[pl.ds(loc, 16)]
                rv = rwin[pl.ds(loc, 16)]
                cv = cwin[pl.ds(loc, 16)]
                cell = (bv * GS + rv) * GS + cv
                binv = cell >> SH
                pidx = io * NBIN + binv
                pv = plsc.load_gather(ptr_v, [pidx])
                plsc.store_scatter(ptr_v, [pidx], pv + 1)
                loc2 = 2 * (loc + io)
                r0 = plsc.bitcast(plsc.load_gather(regwin, [loc2]),
                                  jnp.int32)
                r1 = plsc.bitcast(plsc.load_gather(regwin, [loc2 + 1]),
                                  jnp.int32)
                cellbuf[pl.ds(loc, 16)] = cell & (CPB - 1)
                ibuf[pl.ds(loc, 16)] = base + loc + io
                r0buf[pl.ds(loc, 16)] = r0
                r1buf[pl.ds(loc, 16)] = r1
                destbuf[pl.ds(loc, 16)] = pv
                return c2

            lax.fori_loop(0, WB // 16, vr, 0)

            d1 = pltpu.async_copy(cellbuf, cell_hbm.at[destbuf], sem)
            d2 = pltpu.async_copy(ibuf, i_hbm.at[destbuf], sem)
            d3 = pltpu.async_copy(r0buf, r0_hbm.at[destbuf], sem)
            d4 = pltpu.async_copy(r1buf, r1_hbm.at[destbuf], sem)
            d1.wait()
            d2.wait()
            d3.wait()
            d4.wait()
            return carry

        lax.fori_loop(0, PTS_W // WB, win, 0)

    @functools.partial(
        pl.kernel,
        out_type=jax.ShapeDtypeStruct((NB * GS * GS * 2,), jnp.float32),
        mesh=mesh,
        compiler_params=params,
        interpret=interpret,
        scratch_types=[
            pltpu.VMEM((CPB,), jnp.int32),
            pltpu.VMEM((2 * CPB,), jnp.float32),
            pltpu.VMEM((WC,), jnp.int32),
            pltpu.VMEM((WC,), jnp.int32),
            pltpu.VMEM((WC,), jnp.int32),
            pltpu.VMEM((WC,), jnp.int32),
            pltpu.VMEM((2, 16), jnp.int32),
            pltpu.VMEM((2 * WC,), jnp.float32),
        ],
    )
    def phase_c(cell_hbm, i_hbm, r0_hbm, r1_hbm, seg_hbm, conf_hbm,
                win_v, ev, cwin, iwin, r0win, r1win, segw, outw):
        wid = _wid()
        io = iota()
        zero16f = jnp.zeros((16,), jnp.float32)
        neg1 = jnp.zeros((16,), jnp.int32) - 1

        pltpu.sync_copy(seg_hbm.at[wid], segw)
        start = pl.multiple_of(jnp.max(segw[0, :]), 8)
        cnt = jnp.max(segw[1, :])

        def zw(k, carry):
            win_v[pl.ds(k * 16, 16)] = neg1
            return carry

        lax.fori_loop(0, CPB // 16, zw, 0)

        def ze(k, carry):
            ev[pl.ds(k * 16, 16)] = zero16f
            return carry

        lax.fori_loop(0, 2 * CPB // 16, ze, 0)

        nwin = (cnt + WC - 1) // WC

        def c1(g, carry):
            pltpu.sync_copy(cell_hbm.at[pl.ds(start + g * WC, WC)], cwin)
            pltpu.sync_copy(i_hbm.at[pl.ds(start + g * WC, WC)], iwin)

            def vr(k, c2):
                loc = k * 16
                valid = (g * WC + loc + io) < cnt
                cellv = cwin[pl.ds(loc, 16)] & (CPB - 1)
                iv = iwin[pl.ds(loc, 16)]
                ivs = jnp.where(valid, iv, -1)
                cur = plsc.load_gather(win_v, [cellv])

                def cond(cu):
                    return jnp.any(ivs > cu)

                def body(cu):
                    plsc.store_scatter(win_v, [cellv], ivs, mask=ivs > cu)
                    return plsc.load_gather(win_v, [cellv])

                lax.while_loop(cond, body, cur)
                return c2

            return lax.fori_loop(0, WC // 16, vr, carry)

        lax.fori_loop(0, nwin, c1, 0)

        def c2p(g, carry):
            pltpu.sync_copy(cell_hbm.at[pl.ds(start + g * WC, WC)], cwin)
            pltpu.sync_copy(i_hbm.at[pl.ds(start + g * WC, WC)], iwin)
            pltpu.sync_copy(r0_hbm.at[pl.ds(start + g * WC, WC)], r0win)
            pltpu.sync_copy(r1_hbm.at[pl.ds(start + g * WC, WC)], r1win)

            def vr(k, c2):
                loc = k * 16
                valid = (g * WC + loc + io) < cnt
                cellv = cwin[pl.ds(loc, 16)] & (CPB - 1)
                iv = iwin[pl.ds(loc, 16)]
                ivs = jnp.where(valid, iv, -1)
                r0 = plsc.bitcast(r0win[pl.ds(loc, 16)], jnp.float32)
                r1 = plsc.bitcast(r1win[pl.ds(loc, 16)], jnp.float32)
                w = plsc.load_gather(win_v, [cellv])
                mok = (ivs == w) & valid
                plsc.store_scatter(ev, [cellv * 2], r0, mask=mok)
                plsc.store_scatter(ev, [cellv * 2 + 1], r1, mask=mok)
                return c2

            return lax.fori_loop(0, WC // 16, vr, carry)

        lax.fori_loop(0, nwin, c2p, 0)

        def fw(g, carry):
            def vr(k, c2):
                cc = (g * WC + k * 16 + io) * 2
                v0 = plsc.load_gather(ev, [cc])
                v1 = plsc.load_gather(ev, [cc + 1])
                m = jnp.maximum(v0, v1)
                e0 = jnp.exp(v0 - m)
                e1 = jnp.exp(v1 - m)
                s = e0 + e1
                lo = (k * 16 + io) * 2
                plsc.store_scatter(outw, [lo], e0 / s)
                plsc.store_scatter(outw, [lo + 1], e1 / s)
                return c2

            lax.fori_loop(0, WC // 16, vr, 0)
            pltpu.sync_copy(
                outw,
                conf_hbm.at[pl.ds(wid * 2 * CPB + g * 2 * WC, 2 * WC)])
            return carry

        lax.fori_loop(0, CPB // WC, fw, 0)

    def run(reg, batch_idx, row_idx, col_idx, n, debug=False):
        pad = N_P - n
        i32 = jnp.int32
        b_p = jnp.concatenate([batch_idx.astype(i32),
                               jnp.full((pad,), NB, i32)])
        r_p = jnp.concatenate([row_idx.astype(i32),
                               jnp.zeros((pad,), i32)])
        c_p = jnp.concatenate([col_idx.astype(i32),
                               jnp.zeros((pad,), i32)])
        reg_p = jnp.concatenate([reg, jnp.zeros((pad, 2), reg.dtype)])

        counts = phase_a(b_p, r_p)
        cnts = counts.reshape(NW, NLANE, NBIN)
        per_bin = cnts.transpose(2, 0, 1).reshape(NBIN, NW * NLANE)
        tot = per_bin.sum(axis=1)
        tot_pad = ((tot + 7) // 8) * 8
        segstart = jnp.concatenate(
            [jnp.zeros((1,), i32),
             jnp.cumsum(tot_pad).astype(i32)])[:NBIN]
        within = jnp.cumsum(per_bin, axis=1).astype(i32) - per_bin
        bases = (segstart[:, None] + within).reshape(NBIN, NW, NLANE)
        bases = bases.transpose(1, 2, 0).reshape(NW, NLANE * NBIN)

        cell_a, i_a, r0_a, r1_a = phase_b(
            b_p, r_p, c_p, reg_p.reshape(-1), bases)

        seg = jnp.stack([segstart[:NW], tot[:NW].astype(i32)], axis=1)
        seg = jnp.broadcast_to(seg[:, :, None], (NW, 2, 16)).astype(i32)
        conf_flat = phase_c(cell_a, i_a, r0_a, r1_a, seg)
        if debug:
            return dict(counts=counts, bases=bases, seg=seg,
                        binned=jnp.stack([cell_a, i_a, r0_a, r1_a], 1),
                        conf=conf_flat.reshape(NB, GS, GS, 2))
        return conf_flat.reshape(NB, GS, GS, 2)

    return run


_run = _build()


def kernel(reg, batch_idx, row_idx, col_idx):
    return _run(reg, batch_idx, row_idx, col_idx, N)


# speedup vs baseline: 2.5287x; 2.5287x over previous
import functools

import jax
import jax.numpy as jnp
from jax import lax
from jax.experimental import pallas as pl
from jax.experimental.pallas import tpu as pltpu
from jax.experimental.pallas import tpu_sc as plsc

N = 2_000_000


def _build(NB=4, GS=512, NW=32, PTS_W=65_536, WA=2048, WB=1024, WC=512,
           CHUNK=128, interpret=False):
    N_P = NW * PTS_W
    CELLS = NB * GS * GS
    CPB = CELLS // NW
    SH = CPB.bit_length() - 1
    SHA = SH - (GS.bit_length() - 1)
    NLANE = 16
    NBIN = NW + 1
    WF = min(2048, CPB)
    HALF = N_P // 2 + 8 * NBIN + WC
    TOT = 2 * HALF

    mesh = plsc.VectorSubcoreMesh(core_axis_name="c", subcore_axis_name="s",
                                  num_cores=2, num_subcores=16)
    params = pltpu.CompilerParams(needs_layout_passes=False,
                                  use_tc_tiling_on_sc=False)

    def _wid():
        return lax.axis_index("c") * 16 + lax.axis_index("s")

    iota = lambda: lax.iota(jnp.int32, 16)

    @functools.partial(
        pl.kernel,
        out_type=jax.ShapeDtypeStruct((NW, NLANE * NBIN), jnp.int32),
        mesh=mesh,
        compiler_params=params,
        interpret=interpret,
        scratch_types=[
            pltpu.VMEM((NLANE * NBIN,), jnp.int32),
            pltpu.VMEM((WA,), jnp.int32),
            pltpu.VMEM((WA,), jnp.int32),
        ],
    )
    def phase_a(b_hbm, r_hbm, counts_hbm, cnt_v, bwin, rwin):
        wid = _wid()
        io = iota()
        zero16 = jnp.zeros((16,), jnp.int32)

        def z(k, carry):
            cnt_v[pl.ds(k * 16, 16)] = zero16
            return carry

        lax.fori_loop(0, NBIN, z, 0)

        def win(g, carry):
            base = wid * PTS_W + g * WA
            pltpu.sync_copy(b_hbm.at[pl.ds(base, WA)], bwin)
            pltpu.sync_copy(r_hbm.at[pl.ds(base, WA)], rwin)

            def vr(k, c2):
                bv = bwin[pl.ds(k * 16, 16)]
                rv = rwin[pl.ds(k * 16, 16)]
                binv = (bv * GS + rv) >> SHA
                pidx = io * NBIN + binv
                cur = plsc.load_gather(cnt_v, [pidx])
                plsc.store_scatter(cnt_v, [pidx], cur + 1)
                return c2

            return lax.fori_loop(0, WA // 16, vr, carry)

        lax.fori_loop(0, PTS_W // WA, win, 0)
        pltpu.sync_copy(cnt_v, counts_hbm.at[wid])

    @functools.partial(
        pl.kernel,
        out_type=(
            jax.ShapeDtypeStruct((TOT, 16), jnp.int32),
            jax.ShapeDtypeStruct((2 * CELLS,), jnp.int32),
            jax.ShapeDtypeStruct((2 * 2 * CELLS,), jnp.float32),
        ),
        mesh=mesh,
        compiler_params=params,
        interpret=interpret,
        scratch_types=[
            pltpu.VMEM((NLANE * NBIN,), jnp.int32),
            pltpu.VMEM((WB,), jnp.int32),
            pltpu.VMEM((WB,), jnp.int32),
            pltpu.VMEM((WB,), jnp.int32),
            pltpu.VMEM((2 * WB,), jnp.float32),
            pltpu.VMEM((WB, 16), jnp.int32),
            pltpu.VMEM((WB // CHUNK, CHUNK), jnp.int32),
            pltpu.VMEM((CPB,), jnp.int32),
            pltpu.VMEM((2 * CPB,), jnp.float32),
            pltpu.VMEM((WC, 16), jnp.int32),
            pltpu.VMEM((2, 2, 16), jnp.int32),
            pltpu.SemaphoreType.DMA,
        ],
    )
    def phase_bc(b_hbm, r_hbm, c_hbm, reg_hbm, bases_hbm, seg_hbm,
                 binned_hbm, win_hbm, ev_hbm,
                 ptr_v, bwin, rwin, cwin, regwin, recbuf, destb,
                 win_v, ev, recwin, segw, sem):
        wid = _wid()
        sc = lax.axis_index("c")
        t = lax.axis_index("s")
        io = iota()
        zero16f = jnp.zeros((16,), jnp.float32)
        neg1 = jnp.zeros((16,), jnp.int32) - 1
        col0 = jnp.zeros((16,), jnp.int32)
        col1 = col0 + 1
        col2 = col0 + 2
        col3 = col0 + 3

        pltpu.sync_copy(bases_hbm.at[wid], ptr_v)
        pltpu.sync_copy(seg_hbm.at[wid], segw)

        def win(g, carry):
            base = wid * PTS_W + g * WB
            pltpu.sync_copy(b_hbm.at[pl.ds(base, WB)], bwin)
            pltpu.sync_copy(r_hbm.at[pl.ds(base, WB)], rwin)
            pltpu.sync_copy(c_hbm.at[pl.ds(base, WB)], cwin)
            pltpu.sync_copy(reg_hbm.at[pl.ds(2 * base, 2 * WB)], regwin)

            def vr(k, c2):
                loc = k * 16
                lv = loc + io
                bv = bwin[pl.ds(loc, 16)]
                rv = rwin[pl.ds(loc, 16)]
                cv = cwin[pl.ds(loc, 16)]
                cell = (bv * GS + rv) * GS + cv
                binv = cell >> SH
                pidx = io * NBIN + binv
                pv = plsc.load_gather(ptr_v, [pidx])
                plsc.store_scatter(ptr_v, [pidx], pv + 1)
                loc2 = 2 * lv
                r0 = plsc.bitcast(plsc.load_gather(regwin, [loc2]),
                                  jnp.int32)
                r1 = plsc.bitcast(plsc.load_gather(regwin, [loc2 + 1]),
                                  jnp.int32)
                plsc.store_scatter(recbuf, [lv, col0], cell & (CPB - 1))
                plsc.store_scatter(recbuf, [lv, col1], base + lv)
                plsc.store_scatter(recbuf, [lv, col2], r0)
                plsc.store_scatter(recbuf, [lv, col3], r1)
                rowv = col0 + (k // (CHUNK // 16))
                cpos = (k % (CHUNK // 16)) * 16 + io
                plsc.store_scatter(destb, [rowv, cpos], pv)
                return c2

            lax.fori_loop(0, WB // 16, vr, 0)

            def chs(ch, c3):
                pltpu.async_copy(recbuf.at[pl.ds(ch * CHUNK, CHUNK), :],
                                 binned_hbm.at[destb.at[ch]], sem).wait()
                return c3

            return lax.fori_loop(0, WB // CHUNK, chs, carry)

        lax.fori_loop(0, PTS_W // WB, win, 0)

        plsc.subcore_barrier()

        for j in (0, 1):
            binj = 2 * t + j
            start = pl.multiple_of(jnp.max(segw[j, 0, :]), 8)
            cnt = jnp.max(segw[j, 1, :])

            def zw(k, carry):
                win_v[pl.ds(k * 16, 16)] = neg1
                return carry

            lax.fori_loop(0, CPB // 16, zw, 0)

            def ze(k, carry):
                ev[pl.ds(k * 16, 16)] = zero16f
                return carry

            lax.fori_loop(0, 2 * CPB // 16, ze, 0)

            nwin = (cnt + WC - 1) // WC

            def c1(g, carry):
                pltpu.sync_copy(binned_hbm.at[pl.ds(start + g * WC, WC)],
                                recwin)

                def vr(k, c2):
                    lv = k * 16 + io
                    valid = (g * WC + lv) < cnt
                    cellv = plsc.load_gather(recwin, [lv, col0]) & (CPB - 1)
                    iv = plsc.load_gather(recwin, [lv, col1])
                    ivs = jnp.where(valid, iv, -1)
                    cur = plsc.load_gather(win_v, [cellv])

                    def cond(cu):
                        return jnp.any(ivs > cu)

                    def body(cu):
                        plsc.store_scatter(win_v, [cellv], ivs,
                                           mask=ivs > cu)
                        return plsc.load_gather(win_v, [cellv])

                    lax.while_loop(cond, body, cur)
                    return c2

                return lax.fori_loop(0, WC // 16, vr, carry)

            lax.fori_loop(0, nwin, c1, 0)

            def c2p(g, carry):
                pltpu.sync_copy(binned_hbm.at[pl.ds(start + g * WC, WC)],
                                recwin)

                def vr(k, c2):
                    lv = k * 16 + io
                    valid = (g * WC + lv) < cnt
                    cellv = plsc.load_gather(recwin, [lv, col0]) & (CPB - 1)
                    iv = plsc.load_gather(recwin, [lv, col1])
                    ivs = jnp.where(valid, iv, -1)
                    r0 = plsc.bitcast(plsc.load_gather(recwin, [lv, col2]),
                                      jnp.float32)
                    r1 = plsc.bitcast(plsc.load_gather(recwin, [lv, col3]),
                                      jnp.float32)
                    w = plsc.load_gather(win_v, [cellv])
                    mok = (ivs == w) & valid
                    plsc.store_scatter(ev, [cellv * 2], r0, mask=mok)
                    plsc.store_scatter(ev, [cellv * 2 + 1], r1, mask=mok)
                    return c2

                return lax.fori_loop(0, WC // 16, vr, carry)

            lax.fori_loop(0, nwin, c2p, 0)

            pltpu.sync_copy(win_v,
                            win_hbm.at[pl.ds(sc * CELLS + binj * CPB, CPB)])
            pltpu.sync_copy(
                ev,
                ev_hbm.at[pl.ds(2 * sc * CELLS + binj * 2 * CPB, 2 * CPB)])

    @functools.partial(
        pl.kernel,
        out_type=jax.ShapeDtypeStruct((2 * CELLS,), jnp.float32),
        mesh=mesh,
        compiler_params=params,
        interpret=interpret,
        scratch_types=[
            pltpu.VMEM((WF,), jnp.int32),
            pltpu.VMEM((WF,), jnp.int32),
            pltpu.VMEM((2 * WF,), jnp.float32),
            pltpu.VMEM((2 * WF,), jnp.float32),
            pltpu.VMEM((2 * WF,), jnp.float32),
        ],
    )
    def merge(win_hbm, ev_hbm, conf_hbm, wa, wb, ea, eb, outw):
        wid = _wid()
        io = iota()

        def fw(g, carry):
            cbase = wid * CPB + g * WF
            pltpu.sync_copy(win_hbm.at[pl.ds(cbase, WF)], wa)
            pltpu.sync_copy(win_hbm.at[pl.ds(CELLS + cbase, WF)], wb)
            pltpu.sync_copy(ev_hbm.at[pl.ds(2 * cbase, 2 * WF)], ea)
            pltpu.sync_copy(ev_hbm.at[pl.ds(2 * CELLS + 2 * cbase, 2 * WF)],
                            eb)

            def vr(k, c2):
                lv = k * 16 + io
                lv2 = lv * 2
                wav = wa[pl.ds(k * 16, 16)]
                wbv = wb[pl.ds(k * 16, 16)]
                sel = wbv > wav
                v0a = plsc.load_gather(ea, [lv2])
                v1a = plsc.load_gather(ea, [lv2 + 1])
                v0b = plsc.load_gather(eb, [lv2])
                v1b = plsc.load_gather(eb, [lv2 + 1])
                v0 = jnp.where(sel, v0b, v0a)
                v1 = jnp.where(sel, v1b, v1a)
                m = jnp.maximum(v0, v1)
                e0 = jnp.exp(v0 - m)
                e1 = jnp.exp(v1 - m)
                s = e0 + e1
                plsc.store_scatter(outw, [lv2], e0 / s)
                plsc.store_scatter(outw, [lv2 + 1], e1 / s)
                return c2

            lax.fori_loop(0, WF // 16, vr, 0)
            pltpu.sync_copy(outw, conf_hbm.at[pl.ds(2 * cbase, 2 * WF)])
            return carry

        lax.fori_loop(0, CPB // WF, fw, 0)

    def run(reg, batch_idx, row_idx, col_idx, n, debug=False):
        pad = N_P - n
        i32 = jnp.int32
        b_p = jnp.concatenate([batch_idx.astype(i32),
                               jnp.full((pad,), NB, i32)])
        r_p = jnp.concatenate([row_idx.astype(i32),
                               jnp.zeros((pad,), i32)])
        c_p = jnp.concatenate([col_idx.astype(i32),
                               jnp.zeros((pad,), i32)])
        reg_p = jnp.concatenate([reg, jnp.zeros((pad, 2), reg.dtype)])

        counts = phase_a(b_p, r_p)
        cnts = counts.reshape(NW, NLANE, NBIN)
        bases_l, segs_l, tots_l = [], [], []
        for s in (0, 1):
            sub = cnts[s * 16:(s + 1) * 16]
            per_bin = sub.transpose(2, 0, 1).reshape(NBIN, 16 * NLANE)
            tot = per_bin.sum(axis=1)
            tot_pad = ((tot + 7) // 8) * 8
            segstart = s * HALF + jnp.concatenate(
                [jnp.zeros((1,), i32),
                 jnp.cumsum(tot_pad).astype(i32)])[:NBIN]
            within = jnp.cumsum(per_bin, axis=1).astype(i32) - per_bin
            bases = (segstart[:, None] + within).reshape(NBIN, 16, NLANE)
            bases_l.append(bases.transpose(1, 2, 0).reshape(16,
                                                            NLANE * NBIN))
            segs_l.append(segstart[:NW])
            tots_l.append(tot[:NW].astype(i32))
        bases = jnp.concatenate(bases_l, axis=0)

        seg_rows = []
        for s in (0, 1):
            st = segs_l[s].reshape(16, 2)
            ct = tots_l[s].reshape(16, 2)
            seg_rows.append(jnp.stack([st, ct], axis=2))
        seg = jnp.concatenate(seg_rows, axis=0)
        seg = jnp.broadcast_to(seg[:, :, :, None],
                               (NW, 2, 2, 16)).astype(i32)

        _, win_o, ev_o = phase_bc(b_p, r_p, c_p, reg_p.reshape(-1),
                                  bases, seg)
        conf_flat = merge(win_o, ev_o)
        if debug:
            return dict(counts=counts, bases=bases, seg=seg,
                        win=win_o, ev=ev_o,
                        conf=conf_flat.reshape(NB, GS, GS, 2))
        return conf_flat.reshape(NB, GS, GS, 2)

    return run


_run = _build()


def kernel(reg, batch_idx, row_idx, col_idx):
    return _run(reg, batch_idx, row_idx, col_idx, N)


# speedup vs baseline: 2.5298x; 1.0004x over previous
import functools

import jax
import jax.numpy as jnp
from jax import lax
from jax.experimental import pallas as pl
from jax.experimental.pallas import tpu as pltpu
from jax.experimental.pallas import tpu_sc as plsc

N = 2_000_000


def _build(NB=4, GS=512, NW=32, PTS_W=65_536, WA=2048, WB=1024, WC=512,
           CHUNK=128, interpret=False):
    N_P = NW * PTS_W
    CELLS = NB * GS * GS
    CPB = CELLS // NW
    SH = CPB.bit_length() - 1
    SHA = SH - (GS.bit_length() - 1)
    NLANE = 16
    NBIN = NW + 1
    WF = min(2048, CPB)
    HALF = N_P // 2 + 8 * NBIN + WC
    TOT = 2 * HALF

    mesh = plsc.VectorSubcoreMesh(core_axis_name="c", subcore_axis_name="s",
                                  num_cores=2, num_subcores=16)
    params = pltpu.CompilerParams(needs_layout_passes=False,
                                  use_tc_tiling_on_sc=False)

    def _wid():
        return lax.axis_index("c") * 16 + lax.axis_index("s")

    iota = lambda: lax.iota(jnp.int32, 16)

    @functools.partial(
        pl.kernel,
        out_type=jax.ShapeDtypeStruct((NW, NLANE * NBIN), jnp.int32),
        mesh=mesh,
        compiler_params=params,
        interpret=interpret,
        scratch_types=[
            pltpu.VMEM((NLANE * NBIN,), jnp.int32),
            pltpu.VMEM((WA,), jnp.int32),
            pltpu.VMEM((WA,), jnp.int32),
        ],
    )
    def phase_a(b_hbm, r_hbm, counts_hbm, cnt_v, bwin, rwin):
        wid = _wid()
        io = iota()
        zero16 = jnp.zeros((16,), jnp.int32)

        def z(k, carry):
            cnt_v[pl.ds(k * 16, 16)] = zero16
            return carry

        lax.fori_loop(0, NBIN, z, 0)

        def win(g, carry):
            base = wid * PTS_W + g * WA
            pltpu.sync_copy(b_hbm.at[pl.ds(base, WA)], bwin)
            pltpu.sync_copy(r_hbm.at[pl.ds(base, WA)], rwin)

            def vr(k, c2):
                bv = bwin[pl.ds(k * 16, 16)]
                rv = rwin[pl.ds(k * 16, 16)]
                binv = (bv * GS + rv) >> SHA
                pidx = io * NBIN + binv
                cur = plsc.load_gather(cnt_v, [pidx])
                plsc.store_scatter(cnt_v, [pidx], cur + 1)
                return c2

            return lax.fori_loop(0, WA // 16, vr, carry)

        lax.fori_loop(0, PTS_W // WA, win, 0)
        pltpu.sync_copy(cnt_v, counts_hbm.at[wid])

    @functools.partial(
        pl.kernel,
        out_type=(
            jax.ShapeDtypeStruct((2 * CELLS,), jnp.int32),
            jax.ShapeDtypeStruct((2 * 2 * CELLS,), jnp.float32),
        ),
        mesh=mesh,
        compiler_params=params,
        interpret=interpret,
        scratch_types=[
            pltpu.HBM((TOT, 16), jnp.int32),
            pltpu.VMEM((NLANE * NBIN,), jnp.int32),
            pltpu.VMEM((WB,), jnp.int32),
            pltpu.VMEM((WB,), jnp.int32),
            pltpu.VMEM((WB,), jnp.int32),
            pltpu.VMEM((2 * WB,), jnp.float32),
            pltpu.VMEM((WB, 16), jnp.int32),
            pltpu.VMEM((WB // CHUNK, CHUNK), jnp.int32),
            pltpu.VMEM((CPB,), jnp.int32),
            pltpu.VMEM((2 * CPB,), jnp.float32),
            pltpu.VMEM((WC, 16), jnp.int32),
            pltpu.VMEM((2, 2, 16), jnp.int32),
            pltpu.SemaphoreType.DMA,
        ],
    )
    def phase_bc(b_hbm, r_hbm, c_hbm, reg_hbm, bases_hbm, seg_hbm,
                 win_hbm, ev_hbm,
                 binned_hbm, ptr_v, bwin, rwin, cwin, regwin, recbuf, destb,
                 win_v, ev, recwin, segw, sem):
        wid = _wid()
        sc = lax.axis_index("c")
        t = lax.axis_index("s")
        io = iota()
        zero16f = jnp.zeros((16,), jnp.float32)
        neg1 = jnp.zeros((16,), jnp.int32) - 1
        col0 = jnp.zeros((16,), jnp.int32)
        col1 = col0 + 1
        col2 = col0 + 2
        col3 = col0 + 3

        pltpu.sync_copy(bases_hbm.at[wid], ptr_v)
        pltpu.sync_copy(seg_hbm.at[wid], segw)

        def win(g, carry):
            base = wid * PTS_W + g * WB
            pltpu.sync_copy(b_hbm.at[pl.ds(base, WB)], bwin)
            pltpu.sync_copy(r_hbm.at[pl.ds(base, WB)], rwin)
            pltpu.sync_copy(c_hbm.at[pl.ds(base, WB)], cwin)
            pltpu.sync_copy(reg_hbm.at[pl.ds(2 * base, 2 * WB)], regwin)

            def vr(k, c2):
                loc = k * 16
                lv = loc + io
                bv = bwin[pl.ds(loc, 16)]
                rv = rwin[pl.ds(loc, 16)]
                cv = cwin[pl.ds(loc, 16)]
                cell = (bv * GS + rv) * GS + cv
                binv = cell >> SH
                pidx = io * NBIN + binv
                pv = plsc.load_gather(ptr_v, [pidx])
                plsc.store_scatter(ptr_v, [pidx], pv + 1)
                loc2 = 2 * lv
                r0 = plsc.bitcast(plsc.load_gather(regwin, [loc2]),
                                  jnp.int32)
                r1 = plsc.bitcast(plsc.load_gather(regwin, [loc2 + 1]),
                                  jnp.int32)
                plsc.store_scatter(recbuf, [lv, col0], cell & (CPB - 1))
                plsc.store_scatter(recbuf, [lv, col1], base + lv)
                plsc.store_scatter(recbuf, [lv, col2], r0)
                plsc.store_scatter(recbuf, [lv, col3], r1)
                rowv = col0 + (k // (CHUNK // 16))
                cpos = (k % (CHUNK // 16)) * 16 + io
                plsc.store_scatter(destb, [rowv, cpos], pv)
                return c2

            lax.fori_loop(0, WB // 16, vr, 0)

            def chs(ch, c3):
                pltpu.async_copy(recbuf.at[pl.ds(ch * CHUNK, CHUNK), :],
                                 binned_hbm.at[destb.at[ch]], sem).wait()
                return c3

            return lax.fori_loop(0, WB // CHUNK, chs, carry)

        lax.fori_loop(0, PTS_W // WB, win, 0)

        plsc.subcore_barrier()

        for j in (0, 1):
            binj = 2 * t + j
            start = pl.multiple_of(jnp.max(segw[j, 0, :]), 8)
            cnt = jnp.max(segw[j, 1, :])

            def zw(k, carry):
                win_v[pl.ds(k * 16, 16)] = neg1
                return carry

            lax.fori_loop(0, CPB // 16, zw, 0)

            def ze(k, carry):
                ev[pl.ds(k * 16, 16)] = zero16f
                return carry

            lax.fori_loop(0, 2 * CPB // 16, ze, 0)

            nwin = (cnt + WC - 1) // WC

            def c1(g, carry):
                pltpu.sync_copy(binned_hbm.at[pl.ds(start + g * WC, WC)],
                                recwin)

                def vr(k, c2):
                    lv = k * 16 + io
                    valid = (g * WC + lv) < cnt
                    cellv = plsc.load_gather(recwin, [lv, col0]) & (CPB - 1)
                    iv = plsc.load_gather(recwin, [lv, col1])
                    ivs = jnp.where(valid, iv, -1)
                    cur = plsc.load_gather(win_v, [cellv])

                    def cond(cu):
                        return jnp.any(ivs > cu)

                    def body(cu):
                        plsc.store_scatter(win_v, [cellv], ivs,
                                           mask=ivs > cu)
                        return plsc.load_gather(win_v, [cellv])

                    lax.while_loop(cond, body, cur)
                    return c2

                return lax.fori_loop(0, WC // 16, vr, carry)

            lax.fori_loop(0, nwin, c1, 0)

            def c2p(g, carry):
                pltpu.sync_copy(binned_hbm.at[pl.ds(start + g * WC, WC)],
                                recwin)

                def vr(k, c2):
                    lv = k * 16 + io
                    valid = (g * WC + lv) < cnt
                    cellv = plsc.load_gather(recwin, [lv, col0]) & (CPB - 1)
                    iv = plsc.load_gather(recwin, [lv, col1])
                    ivs = jnp.where(valid, iv, -1)
                    r0 = plsc.bitcast(plsc.load_gather(recwin, [lv, col2]),
                                      jnp.float32)
                    r1 = plsc.bitcast(plsc.load_gather(recwin, [lv, col3]),
                                      jnp.float32)
                    w = plsc.load_gather(win_v, [cellv])
                    mok = (ivs == w) & valid
                    plsc.store_scatter(ev, [cellv * 2], r0, mask=mok)
                    plsc.store_scatter(ev, [cellv * 2 + 1], r1, mask=mok)
                    return c2

                return lax.fori_loop(0, WC // 16, vr, carry)

            lax.fori_loop(0, nwin, c2p, 0)

            pltpu.sync_copy(win_v,
                            win_hbm.at[pl.ds(sc * CELLS + binj * CPB, CPB)])
            pltpu.sync_copy(
                ev,
                ev_hbm.at[pl.ds(2 * sc * CELLS + binj * 2 * CPB, 2 * CPB)])

    @functools.partial(
        pl.kernel,
        out_type=jax.ShapeDtypeStruct((2 * CELLS,), jnp.float32),
        mesh=mesh,
        compiler_params=params,
        interpret=interpret,
        scratch_types=[
            pltpu.VMEM((WF,), jnp.int32),
            pltpu.VMEM((WF,), jnp.int32),
            pltpu.VMEM((2 * WF,), jnp.float32),
            pltpu.VMEM((2 * WF,), jnp.float32),
            pltpu.VMEM((2 * WF,), jnp.float32),
        ],
    )
    def merge(win_hbm, ev_hbm, conf_hbm, wa, wb, ea, eb, outw):
        wid = _wid()
        io = iota()

        def fw(g, carry):
            cbase = wid * CPB + g * WF
            pltpu.sync_copy(win_hbm.at[pl.ds(cbase, WF)], wa)
            pltpu.sync_copy(win_hbm.at[pl.ds(CELLS + cbase, WF)], wb)
            pltpu.sync_copy(ev_hbm.at[pl.ds(2 * cbase, 2 * WF)], ea)
            pltpu.sync_copy(ev_hbm.at[pl.ds(2 * CELLS + 2 * cbase, 2 * WF)],
                            eb)

            def vr(k, c2):
                lv = k * 16 + io
                lv2 = lv * 2
                wav = wa[pl.ds(k * 16, 16)]
                wbv = wb[pl.ds(k * 16, 16)]
                sel = wbv > wav
                v0a = plsc.load_gather(ea, [lv2])
                v1a = plsc.load_gather(ea, [lv2 + 1])
                v0b = plsc.load_gather(eb, [lv2])
                v1b = plsc.load_gather(eb, [lv2 + 1])
                v0 = jnp.where(sel, v0b, v0a)
                v1 = jnp.where(sel, v1b, v1a)
                m = jnp.maximum(v0, v1)
                e0 = jnp.exp(v0 - m)
                e1 = jnp.exp(v1 - m)
                s = e0 + e1
                plsc.store_scatter(outw, [lv2], e0 / s)
                plsc.store_scatter(outw, [lv2 + 1], e1 / s)
                return c2

            lax.fori_loop(0, WF // 16, vr, 0)
            pltpu.sync_copy(outw, conf_hbm.at[pl.ds(2 * cbase, 2 * WF)])
            return carry

        lax.fori_loop(0, CPB // WF, fw, 0)

    def run(reg, batch_idx, row_idx, col_idx, n, debug=False):
        pad = N_P - n
        i32 = jnp.int32
        b_p = jnp.concatenate([batch_idx.astype(i32),
                               jnp.full((pad,), NB, i32)])
        r_p = jnp.concatenate([row_idx.astype(i32),
                               jnp.zeros((pad,), i32)])
        c_p = jnp.concatenate([col_idx.astype(i32),
                               jnp.zeros((pad,), i32)])
        reg_p = jnp.concatenate([reg, jnp.zeros((pad, 2), reg.dtype)])

        counts = phase_a(b_p, r_p)
        cnts = counts.reshape(NW, NLANE, NBIN)
        bases_l, segs_l, tots_l = [], [], []
        for s in (0, 1):
            sub = cnts[s * 16:(s + 1) * 16]
            per_bin = sub.transpose(2, 0, 1).reshape(NBIN, 16 * NLANE)
            tot = per_bin.sum(axis=1)
            tot_pad = ((tot + 7) // 8) * 8
            segstart = s * HALF + jnp.concatenate(
                [jnp.zeros((1,), i32),
                 jnp.cumsum(tot_pad).astype(i32)])[:NBIN]
            within = jnp.cumsum(per_bin, axis=1).astype(i32) - per_bin
            bases = (segstart[:, None] + within).reshape(NBIN, 16, NLANE)
            bases_l.append(bases.transpose(1, 2, 0).reshape(16,
                                                            NLANE * NBIN))
            segs_l.append(segstart[:NW])
            tots_l.append(tot[:NW].astype(i32))
        bases = jnp.concatenate(bases_l, axis=0)

        seg_rows = []
        for s in (0, 1):
            st = segs_l[s].reshape(16, 2)
            ct = tots_l[s].reshape(16, 2)
            seg_rows.append(jnp.stack([st, ct], axis=2))
        seg = jnp.concatenate(seg_rows, axis=0)
        seg = jnp.broadcast_to(seg[:, :, :, None],
                               (NW, 2, 2, 16)).astype(i32)

        win_o, ev_o = phase_bc(b_p, r_p, c_p, reg_p.reshape(-1),
                               bases, seg)
        conf_flat = merge(win_o, ev_o)
        if debug:
            return dict(counts=counts, bases=bases, seg=seg,
                        win=win_o, ev=ev_o,
                        conf=conf_flat.reshape(NB, GS, GS, 2))
        return conf_flat.reshape(NB, GS, GS, 2)

    return run


_run = _build()


def kernel(reg, batch_idx, row_idx, col_idx):
    return _run(reg, batch_idx, row_idx, col_idx, N)
